# Initial kernel scaffold; baseline (speedup 1.0000x reference)
import jax
import jax.numpy as jnp
from jax.experimental import pallas as pl


def kernel(features, edge_weights, params, threashold):
    raise NotImplementedError("write your pallas kernel here")


# speedup vs baseline: 89.3453x; 89.3453x over previous
import functools

import jax
import jax.numpy as jnp
from jax import lax
from jax.experimental import pallas as pl
from jax.experimental.pallas import tpu as pltpu

_TJ = 128


def _mm(a, b, dims):
    return lax.dot_general(
        a, b, (dims, ((), ())),
        precision=lax.Precision.HIGHEST,
        preferred_element_type=jnp.float32,
    )


def _attention_tile(jt, cut_ref, we_ref, att_ref, x_ref, ewt_ref,
                    wl_ref, blr_ref, blc_ref, wr_ref, brr_ref,
                    xl_s, xlt_s, xr_s, tj, fout):
    n = x_ref.shape[0]

    @pl.when(jt == 0)
    def _():
        xv = x_ref[...]
        wl = wl_ref[...]
        xl_s[...] = _mm(xv, wl, ((1,), (0,))) + blr_ref[...]
        xlt_s[...] = _mm(wl, xv, ((0,), (1,))) + blc_ref[...]
        xr_s[...] = _mm(xv, wr_ref[...], ((1,), (0,))) + brr_ref[...]

    ewt = ewt_ref[...]
    mask = ewt > cut_ref[0, 0]
    xr_t = xr_s[pl.ds(jt * tj, tj), :]
    ciota = lax.broadcasted_iota(jnp.int32, (fout, 1), 0)

    def cbody(c, acc):
        we_c = we_ref[0, c]
        att_c = att_ref[0, c]
        xl_row = xlt_s[pl.ds(c, 1), :]
        onehot = (ciota == c).astype(jnp.float32)
        xr_col = _mm(xr_t, onehot, ((1,), (0,)))
        z = ewt * we_c + (xl_row + xr_col)
        zl = jnp.maximum(z, 0.2 * z)
        return acc + att_c * zl

    alpha = lax.fori_loop(0, fout, cbody,
                          jnp.zeros((tj, n), jnp.float32))
    neg = jnp.float32(-jnp.inf)
    am = jnp.max(jnp.where(mask, alpha, neg), axis=1, keepdims=True)
    am = jnp.where(am == neg, jnp.float32(0.0), am)
    p = jnp.where(mask, jnp.exp(alpha - am), jnp.float32(0.0))
    den = jnp.sum(p, axis=1, keepdims=True)
    a = p / (den + jnp.float32(1e-16))
    return a


def _gat_norm_body(cut_ref, we_ref, att_ref, x_ref, ewt_ref,
                   wl_ref, blr_ref, blc_ref, wr_ref, brr_ref, bias_ref,
                   o_ref, xl_s, xlt_s, xr_s, *, tj, fout):
    jt = pl.program_id(0)
    a = _attention_tile(jt, cut_ref, we_ref, att_ref, x_ref, ewt_ref,
                        wl_ref, blr_ref, blc_ref, wr_ref, brr_ref,
                        xl_s, xlt_s, xr_s, tj, fout)
    out = _mm(a, xl_s[...], ((1,), (0,))) + bias_ref[...]
    mu = jnp.mean(out, axis=1, keepdims=True)
    d0 = out - mu
    var = jnp.sum(d0 * d0, axis=1, keepdims=True) / jnp.float32(fout - 1)
    o_ref[...] = d0 / jnp.sqrt(var)


def _gat_head_body(cut_ref, we_ref, att_ref, x_ref, ewt_ref,
                   wl_ref, blr_ref, blc_ref, wr_ref, brr_ref, bias_ref,
                   l1w_ref, l1b_ref, l2w_ref, l2b_ref,
                   o_ref, xl_s, xlt_s, xr_s, hsum_s, *, tj, fout, nt):
    jt = pl.program_id(0)
    a = _attention_tile(jt, cut_ref, we_ref, att_ref, x_ref, ewt_ref,
                        wl_ref, blr_ref, blc_ref, wr_ref, brr_ref,
                        xl_s, xlt_s, xr_s, tj, fout)
    out = _mm(a, xl_s[...], ((1,), (0,))) + bias_ref[...]

    @pl.when(jt == 0)
    def _():
        hsum_s[...] = jnp.zeros_like(hsum_s)

    hsum_s[...] += jnp.sum(out, axis=0, keepdims=True)

    @pl.when(jt == nt - 1)
    def _():
        n = x_ref.shape[0]
        x5 = hsum_s[...] / jnp.float32(n)
        x6 = jnp.maximum(_mm(x5, l1w_ref[...], ((1,), (0,))) + l1b_ref[...],
                         jnp.float32(0.0))
        o_ref[...] = _mm(x6, l2w_ref[...], ((1,), (0,))) + l2b_ref[...]


def _gat_layer(x, ewt, cut, Wl, bl, Wr, br, We, att, bias, head=None):
    n, _ = x.shape
    fout = Wl.shape[1]
    nt = n // _TJ
    smem = pl.BlockSpec(memory_space=pltpu.SMEM)

    def vm(shape, imap):
        return pl.BlockSpec(shape, imap)

    common_specs = [
        smem,
        smem,
        smem,
        vm(x.shape, lambda j: (0, 0)),
        vm((_TJ, n), lambda j: (j, 0)),
        vm(Wl.shape, lambda j: (0, 0)),
        vm((1, fout), lambda j: (0, 0)),
        vm((fout, 1), lambda j: (0, 0)),
        vm(Wr.shape, lambda j: (0, 0)),
        vm((1, fout), lambda j: (0, 0)),
        vm((1, fout), lambda j: (0, 0)),
    ]
    scratch = [
        pltpu.VMEM((n, fout), jnp.float32),
        pltpu.VMEM((fout, n), jnp.float32),
        pltpu.VMEM((n, fout), jnp.float32),
    ]
    args = [cut, We, att.reshape(1, fout), x, ewt,
            Wl, bl.reshape(1, fout), bl.reshape(fout, 1),
            Wr, br.reshape(1, fout), bias.reshape(1, fout)]

    if head is None:
        return pl.pallas_call(
            functools.partial(_gat_norm_body, tj=_TJ, fout=fout),
            grid=(nt,),
            in_specs=common_specs,
            out_specs=vm((_TJ, fout), lambda j: (j, 0)),
            out_shape=jax.ShapeDtypeStruct((n, fout), jnp.float32),
            scratch_shapes=scratch,
        )(*args)

    l1W, l1b, l2W, l2b = head
    specs = common_specs + [
        vm(l1W.shape, lambda j: (0, 0)),
        vm((1, l1W.shape[1]), lambda j: (0, 0)),
        vm(l2W.shape, lambda j: (0, 0)),
        vm((1, 1), lambda j: (0, 0)),
    ]
    args += [l1W, l1b.reshape(1, -1), l2W, l2b.reshape(1, 1)]
    return pl.pallas_call(
        functools.partial(_gat_head_body, tj=_TJ, fout=fout, nt=nt),
        grid=(nt,),
        in_specs=specs,
        out_specs=vm((1, 1), lambda j: (0, 0)),
        out_shape=jax.ShapeDtypeStruct((1, 1), jnp.float32),
        scratch_shapes=scratch + [pltpu.VMEM((1, fout), jnp.float32)],
    )(*args)


def kernel(features, edge_weights, params, threashold):
    x = jnp.squeeze(features).astype(jnp.float32)
    ew = jnp.squeeze(edge_weights).astype(jnp.float32)
    ewt = ew.T
    cut = (1.0 / threashold) * jnp.ones((1, 1), jnp.float32)

    def layer(name, xin, head=None):
        p = params
        return _gat_layer(xin, ewt, cut,
                          p[name + "_Wl"], p[name + "_bl"],
                          p[name + "_Wr"], p[name + "_br"],
                          p[name + "_We"], p[name + "_att"],
                          p[name + "_bias"], head=head)

    x1 = layer("conv1", x)
    x2 = layer("conv2", x1)
    x3 = layer("conv3", x2)
    out = layer("conv4", x3,
                head=(params["l1_W"], params["l1_b"],
                      params["l2_W"], params["l2_b"]))
    return out.reshape(1)
